# baseline (speedup 1.0000x reference)
import functools

import jax
import jax.numpy as jnp
from jax import lax
from jax.experimental import pallas as pl
from jax.experimental.pallas import tpu as pltpu
from jax.experimental.pallas import tpu_sc as plsc

B, E, K, D, C = 64, 64, 8, 1024, 100
E_BLK = 16
N_STEPS = E // E_BLK
TANH_FACTOR = 10.0

_NC, _NS = 2, 16
_NW = _NC * _NS
_RPW = B // _NW


def _cos_kernel(x_ref, keys_ref, cos_ref):
    x = x_ref[...]
    norm = jnp.sqrt(jnp.sum(x * x, axis=1, keepdims=True))
    xn = x / jnp.maximum(norm, 1e-12)
    cos_ref[...] = jax.lax.dot_general(xn, keys_ref[...],
                                       (((1,), (1,)), ((), ())),
                                       preferred_element_type=jnp.float32)


def _sc_topk(cos_hbm, w_hbm, row_v, w_v, tmpf_v, tmpi_v):
    wid = lax.axis_index("s") * _NC + lax.axis_index("c")
    base = wid * _RPW
    pltpu.sync_copy(cos_hbm.at[pl.ds(base, _RPW)], row_v)
    idx16 = lax.broadcasted_iota(jnp.int32, (16,), 0)
    last = jnp.full((16,), 15, jnp.int32)
    nv = E // 16
    for b in range(_RPW):
        vs = [row_v[b, pl.ds(j * 16, 16)] for j in range(nv)]
        gidx = [idx16 + 16 * j for j in range(nv)]
        zf = jnp.zeros((16,), jnp.float32)
        zi = jnp.zeros((16,), jnp.int32)
        sims = []
        es = []
        for k in range(K):
            mm = vs[0]
            for v in vs[1:]:
                mm = jnp.maximum(mm, v)
            lanes = [mm[l] for l in range(16)]
            while len(lanes) > 1:
                lanes = [jnp.maximum(lanes[2 * t], lanes[2 * t + 1])
                         for t in range(len(lanes) // 2)]
            rm_s = lanes[0]
            rm = zf + rm_s
            cand = jnp.full((16,), 2 * E, jnp.int32)
            for j, v in enumerate(vs):
                cand = jnp.minimum(cand, jnp.where(v == rm, gidx[j], 2 * E))
            cl = [cand[l] for l in range(16)]
            while len(cl) > 1:
                cl = [jnp.minimum(cl[2 * t], cl[2 * t + 1])
                      for t in range(len(cl) // 2)]
            e_s = cl[0]
            e = zi + e_s
            es.append(e_s)
            sims.append(rm_s)
            vs = [jnp.where(gidx[j] == e, -1e30, v)
                  for j, v in enumerate(vs)]
        den = sims[0]
        for s in sims[1:]:
            den = den + s
        den_v = zf + den
        ws = [(zf + s) / den_v for s in sims]
        se = list(es)
        for (a, c) in ((0, 1), (2, 3), (4, 5), (6, 7),
                       (0, 2), (1, 3), (4, 6), (5, 7),
                       (1, 2), (5, 6), (0, 4), (3, 7),
                       (1, 5), (2, 6),
                       (1, 4), (3, 6),
                       (2, 4), (3, 5),
                       (3, 4)):
            lo = jnp.minimum(se[a], se[c])
            hi = jnp.maximum(se[a], se[c])
            se[a], se[c] = lo, hi
        for j in range(nv):
            wj = zf
            for k in range(K):
                wj = jnp.where(gidx[j] == zi + se[k], ws[k], wj)
            w_v[b, pl.ds(j * 16, 16)] = wj
    pltpu.sync_copy(w_v, w_hbm.at[pl.ds(base, _RPW)])


def _ens_kernel(x_ref, keys_ref, w_ref, ew_ref, eb_ref, vw_ref, vb_ref,
                tw_ref, tb_ref, ens_ref, tanh_ref, van_ref, acc_ref):
    i = pl.program_id(0)

    @pl.when(i == 0)
    def _heads():
        x0 = x_ref[...]
        acc_ref[...] = jnp.zeros((B, C), jnp.float32)
        v = jax.lax.dot_general(x0, vw_ref[...], (((1,), (1,)), ((), ())),
                                preferred_element_type=jnp.float32) + vb_ref[...]
        m2 = jnp.max(v, axis=1, keepdims=True)
        s = v - m2
        lse = jnp.log(jnp.sum(jnp.exp(s), axis=1, keepdims=True))
        van_ref[...] = s - lse
        th = jax.lax.dot_general(x0, tw_ref[...], (((1,), (1,)), ((), ())),
                                 preferred_element_type=jnp.float32) + tb_ref[...]
        tanh_ref[...] = jnp.tanh(th * (1.0 / TANH_FACTOR)) * TANH_FACTOR

    x = x_ref[...]
    w = w_ref[...]
    idxs = jax.lax.broadcasted_iota(jnp.int32, (B, E), 1)
    acc = acc_ref[...]
    for j in range(E_BLK):
        e_idx = i * E_BLK + j
        wj = ew_ref[j]
        y = jax.lax.dot_general(x, wj, (((1,), (1,)), ((), ())),
                                preferred_element_type=jnp.float32)
        y = y + eb_ref[j][None, :]
        t = jnp.tanh(y * (1.0 / TANH_FACTOR)) * TANH_FACTOR
        wcol = jnp.sum(jnp.where(idxs == e_idx, w, 0.0), axis=1, keepdims=True)
        acc = acc + wcol * t
    acc_ref[...] = acc

    @pl.when(i == N_STEPS - 1)
    def _finish():
        ens_ref[...] = acc_ref[...]


def _run(x, keys, expert_W, expert_b, vanilla_W, vb2, tanh_W, tb2):
    cos = pl.pallas_call(
        _cos_kernel,
        out_shape=jax.ShapeDtypeStruct((B, E), jnp.float32),
    )(x, keys)

    mesh = plsc.VectorSubcoreMesh(core_axis_name="c", subcore_axis_name="s",
                                  num_cores=_NC)
    w = functools.partial(
        pl.kernel, mesh=mesh,
        out_type=jax.ShapeDtypeStruct((B, E), jnp.float32),
        scratch_types=[pltpu.VMEM((_RPW, E), jnp.float32),
                       pltpu.VMEM((_RPW, E), jnp.float32),
                       pltpu.VMEM((16,), jnp.float32),
                       pltpu.VMEM((16,), jnp.int32)],
    )(_sc_topk)(cos)

    return pl.pallas_call(
        _ens_kernel,
        grid=(N_STEPS,),
        in_specs=[
            pl.BlockSpec((B, D), lambda i: (0, 0)),
            pl.BlockSpec((E, D), lambda i: (0, 0)),
            pl.BlockSpec((B, E), lambda i: (0, 0)),
            pl.BlockSpec((E_BLK, C, D), lambda i: (i, 0, 0)),
            pl.BlockSpec((E_BLK, C), lambda i: (i, 0)),
            pl.BlockSpec((C, D), lambda i: (0, 0)),
            pl.BlockSpec((1, C), lambda i: (0, 0)),
            pl.BlockSpec((C, D), lambda i: (0, 0)),
            pl.BlockSpec((1, C), lambda i: (0, 0)),
        ],
        out_specs=[
            pl.BlockSpec((B, C), lambda i: (0, 0)),
            pl.BlockSpec((B, C), lambda i: (0, 0)),
            pl.BlockSpec((B, C), lambda i: (0, 0)),
        ],
        out_shape=[
            jax.ShapeDtypeStruct((B, C), jnp.float32),
            jax.ShapeDtypeStruct((B, C), jnp.float32),
            jax.ShapeDtypeStruct((B, C), jnp.float32),
        ],
        scratch_shapes=[
            pltpu.VMEM((B, C), jnp.float32),
        ],
    )(x, keys, w, expert_W, expert_b, vanilla_W, vb2, tanh_W, tb2)


def kernel(x, keys, expert_W, expert_b, vanilla_W, vanilla_b, tanh_W, tanh_b,
           x_is_encoded=1):
    ens, tanh_out, van = _run(x, keys, expert_W, expert_b,
                              vanilla_W, vanilla_b.reshape(1, C),
                              tanh_W, tanh_b.reshape(1, C))
    return (ens, tanh_out, van)


# speedup vs baseline: 1.3391x; 1.3391x over previous
import jax
import jax.numpy as jnp
from jax.experimental import pallas as pl
from jax.experimental.pallas import tpu as pltpu

B, E, K, D, C = 64, 64, 8, 1024, 100
E_BLK = 16
N_STEPS = E // E_BLK
TANH_FACTOR = 10.0


def _ens_kernel(x_ref, keys_ref, ew_ref, eb_ref, vw_ref, vb_ref, tw_ref, tb_ref,
                ens_ref, tanh_ref, van_ref,
                w_ref, denom_ref, acc_ref):
    i = pl.program_id(0)

    @pl.when(i == 0)
    def _routing():
        x = x_ref[...]
        norm = jnp.sqrt(jnp.sum(x * x, axis=1, keepdims=True))
        xn = x / jnp.maximum(norm, 1e-12)
        cos = jax.lax.dot_general(xn, keys_ref[...], (((1,), (1,)), ((), ())),
                                  preferred_element_type=jnp.float32)
        idxs = jax.lax.broadcasted_iota(jnp.int32, (B, E), 1)
        work = cos
        sel = jnp.zeros((B, E), dtype=jnp.bool_)
        sims = []
        for _ in range(K):
            m = jnp.max(work, axis=1, keepdims=True)
            is_max = work == m
            first_idx = jnp.min(jnp.where(is_max, idxs, E), axis=1, keepdims=True)
            first = idxs == first_idx
            sel = jnp.logical_or(sel, first)
            sims.append(m)
            work = jnp.where(first, -1e30, work)
        sel_f = sel.astype(jnp.float32)
        row = jax.lax.broadcasted_iota(jnp.int32, (E, E), 0)
        col = jax.lax.broadcasted_iota(jnp.int32, (E, E), 1)
        tri = (row < col).astype(jnp.float32)
        pos = jax.lax.dot_general(sel_f, tri, (((1,), (0,)), ((), ())),
                                  preferred_element_type=jnp.float32)
        w = jnp.zeros((B, E), dtype=jnp.float32)
        den = jnp.zeros((B, 1), dtype=jnp.float32)
        for k in range(K):
            w = jnp.where(jnp.logical_and(sel, pos == float(k)), sims[k], w)
            den = den + sims[k]
        w_ref[...] = w
        denom_ref[...] = den
        acc_ref[...] = jnp.zeros((B, C), jnp.float32)
        v = jax.lax.dot_general(x, vw_ref[...], (((1,), (1,)), ((), ())),
                                preferred_element_type=jnp.float32) + vb_ref[...]
        m2 = jnp.max(v, axis=1, keepdims=True)
        s = v - m2
        lse = jnp.log(jnp.sum(jnp.exp(s), axis=1, keepdims=True))
        van_ref[...] = s - lse
        th = jax.lax.dot_general(x, tw_ref[...], (((1,), (1,)), ((), ())),
                                 preferred_element_type=jnp.float32) + tb_ref[...]
        tanh_ref[...] = jnp.tanh(th * (1.0 / TANH_FACTOR)) * TANH_FACTOR

    x = x_ref[...]
    w = w_ref[...]
    idxs = jax.lax.broadcasted_iota(jnp.int32, (B, E), 1)
    acc = acc_ref[...]
    for j in range(E_BLK):
        e_idx = i * E_BLK + j
        wj = ew_ref[j]
        y = jax.lax.dot_general(x, wj, (((1,), (1,)), ((), ())),
                                preferred_element_type=jnp.float32)
        y = y + eb_ref[j][None, :]
        t = jnp.tanh(y * (1.0 / TANH_FACTOR)) * TANH_FACTOR
        wcol = jnp.sum(jnp.where(idxs == e_idx, w, 0.0), axis=1, keepdims=True)
        acc = acc + wcol * t
    acc_ref[...] = acc

    @pl.when(i == N_STEPS - 1)
    def _finish():
        ens_ref[...] = acc_ref[...] / denom_ref[...]


def _run(x, keys, expert_W, expert_b, vanilla_W, vb2, tanh_W, tb2):
    return pl.pallas_call(
        _ens_kernel,
        grid=(N_STEPS,),
        in_specs=[
            pl.BlockSpec((B, D), lambda i: (0, 0)),
            pl.BlockSpec((E, D), lambda i: (0, 0)),
            pl.BlockSpec((E_BLK, C, D), lambda i: (i, 0, 0)),
            pl.BlockSpec((E_BLK, C), lambda i: (i, 0)),
            pl.BlockSpec((C, D), lambda i: (0, 0)),
            pl.BlockSpec((1, C), lambda i: (0, 0)),
            pl.BlockSpec((C, D), lambda i: (0, 0)),
            pl.BlockSpec((1, C), lambda i: (0, 0)),
        ],
        out_specs=[
            pl.BlockSpec((B, C), lambda i: (0, 0)),
            pl.BlockSpec((B, C), lambda i: (0, 0)),
            pl.BlockSpec((B, C), lambda i: (0, 0)),
        ],
        out_shape=[
            jax.ShapeDtypeStruct((B, C), jnp.float32),
            jax.ShapeDtypeStruct((B, C), jnp.float32),
            jax.ShapeDtypeStruct((B, C), jnp.float32),
        ],
        scratch_shapes=[
            pltpu.VMEM((B, E), jnp.float32),
            pltpu.VMEM((B, 1), jnp.float32),
            pltpu.VMEM((B, C), jnp.float32),
        ],
    )(x, keys, expert_W, expert_b, vanilla_W, vb2, tanh_W, tb2)


def kernel(x, keys, expert_W, expert_b, vanilla_W, vanilla_b, tanh_W, tanh_b,
           x_is_encoded=1):
    ens, tanh_out, van = _run(x, keys, expert_W, expert_b,
                              vanilla_W, vanilla_b.reshape(1, C),
                              tanh_W, tanh_b.reshape(1, C))
    return (ens, tanh_out, van)
